# Initial kernel scaffold; baseline (speedup 1.0000x reference)
import jax
import jax.numpy as jnp
from jax.experimental import pallas as pl


def kernel(x, edge_attr, params, edge_index):
    raise NotImplementedError("write your pallas kernel here")


# speedup vs baseline: 2.4807x; 2.4807x over previous
import functools

import jax
import jax.numpy as jnp
from jax import lax
from jax.experimental import pallas as pl
from jax.experimental.pallas import tpu as pltpu
from jax.experimental.pallas import tpu_sc as plsc

_N = 50000
_E = 800000
_EP = 802816
_H = 64
_GW = 128
_NST = _EP // _GW
_NC = 2
_NS = 16
_NW = _NC * _NS
_NPAD = 50432
_CH = 3584
_NCH = _EP // _CH
_TE = 4096
_TN = 2048

f32 = jnp.float32


def _lnk(h, g, b):
    m = jnp.mean(h, axis=-1, keepdims=True)
    d = h - m
    v = jnp.mean(d * d, axis=-1, keepdims=True)
    return d * lax.rsqrt(v + 1e-5) * g + b


def _geluk(h):
    return 0.5 * h * (1.0 + lax.erf(h * 0.7071067811865476))


def _rep(v):
    return v.reshape(1, -1)


def _pack_pairs(e1):
    u = lax.bitcast_convert_type(e1, jnp.uint32)
    hi = (u[:, :32] + jnp.uint32(0x8000)) & jnp.uint32(0xFFFF0000)
    lo = (u[:, 32:] + jnp.uint32(0x8000)) >> jnp.uint32(16)
    pk = lax.bitcast_convert_type(hi | lo, f32)
    return pk.T


def _pq(x, A_ref, B_ref, lb1_ref):
    P = jnp.dot(x, A_ref[...], preferred_element_type=f32) + lb1_ref[...]
    Q = jnp.dot(x, B_ref[...], preferred_element_type=f32)
    return P, Q


def _enc_call(xin, W, b, g, be, A, B, lb1):
    def body(x_ref, W_ref, b_ref, g_ref, be_ref, A_ref, B_ref, lb1_ref,
             o_ref, p_ref, q_ref):
        h = jnp.dot(x_ref[...], W_ref[...], preferred_element_type=f32) + b_ref[...]
        x0 = _geluk(_lnk(h, g_ref[...], be_ref[...]))
        o_ref[...] = x0
        P, Q = _pq(x0, A_ref, B_ref, lb1_ref)
        p_ref[...] = P
        q_ref[...] = Q

    c0 = lambda i: (0, 0)
    blk = lambda i: (i, 0)
    return pl.pallas_call(
        body,
        grid=((_N + _TN - 1) // _TN,),
        in_specs=[
            pl.BlockSpec((_TN, 8), blk),
            pl.BlockSpec((8, _H), c0),
            pl.BlockSpec((1, _H), c0),
            pl.BlockSpec((1, _H), c0),
            pl.BlockSpec((1, _H), c0),
            pl.BlockSpec((_H, 2 * _H), c0),
            pl.BlockSpec((_H, 2 * _H), c0),
            pl.BlockSpec((1, 2 * _H), c0),
        ],
        out_specs=[
            pl.BlockSpec((_TN, _H), blk),
            pl.BlockSpec((_TN, 2 * _H), blk),
            pl.BlockSpec((_TN, 2 * _H), blk),
        ],
        out_shape=[
            jax.ShapeDtypeStruct((_N, _H), f32),
            jax.ShapeDtypeStruct((_N, 2 * _H), f32),
            jax.ShapeDtypeStruct((_N, 2 * _H), f32),
        ],
    )(xin, W, b, g, be, A, B, lb1)


def _edge_core(e0, Pr, Qc, C, lg1, lbe1, lW2, lb2, lg2, lbe2):
    H = Pr + Qc + jnp.dot(e0, C, preferred_element_type=f32)
    H = _geluk(_lnk(H, lg1, lbe1))
    M = _lnk(jnp.dot(H, lW2, preferred_element_type=f32) + lb2, lg2, lbe2)
    return e0 + M


def _edge_first_call(ea, Pr, Qc, eep, lp):
    def body(ea_ref, pr_ref, qc_ref, W1_ref, b1_ref, g1_ref, be1_ref,
             W2_ref, b2_ref, g2_ref, be2_ref,
             C_ref, lg1_ref, lbe1_ref, lW2_ref, lb2_ref, lg2_ref, lbe2_ref,
             o_ref, pk_ref):
        h0 = _geluk(_lnk(jnp.dot(ea_ref[...], W1_ref[...],
                                 preferred_element_type=f32) + b1_ref[...],
                         g1_ref[...], be1_ref[...]))
        e0 = _lnk(jnp.dot(h0, W2_ref[...], preferred_element_type=f32)
                  + b2_ref[...], g2_ref[...], be2_ref[...])
        e1 = _edge_core(e0, pr_ref[...], qc_ref[...], C_ref[...],
                        lg1_ref[...], lbe1_ref[...], lW2_ref[...],
                        lb2_ref[...], lg2_ref[...], lbe2_ref[...])
        o_ref[...] = e1
        pk_ref[:, 0, :] = _pack_pairs(e1)

    c0 = lambda i: (0, 0)
    blk = lambda i: (i, 0)
    return pl.pallas_call(
        body,
        grid=(_EP // _TE,),
        in_specs=[
            pl.BlockSpec((_TE, 16), blk),
            pl.BlockSpec((_TE, 2 * _H), blk),
            pl.BlockSpec((_TE, 2 * _H), blk),
            pl.BlockSpec((16, 2 * _H), c0),
            pl.BlockSpec((1, 2 * _H), c0),
            pl.BlockSpec((1, 2 * _H), c0),
            pl.BlockSpec((1, 2 * _H), c0),
            pl.BlockSpec((2 * _H, _H), c0),
            pl.BlockSpec((1, _H), c0),
            pl.BlockSpec((1, _H), c0),
            pl.BlockSpec((1, _H), c0),
            pl.BlockSpec((_H, 2 * _H), c0),
            pl.BlockSpec((1, 2 * _H), c0),
            pl.BlockSpec((1, 2 * _H), c0),
            pl.BlockSpec((2 * _H, _H), c0),
            pl.BlockSpec((1, _H), c0),
            pl.BlockSpec((1, _H), c0),
            pl.BlockSpec((1, _H), c0),
        ],
        out_specs=[
            pl.BlockSpec((_TE, _H), blk),
            pl.BlockSpec((32, 1, _TE), lambda i: (0, 0, i)),
        ],
        out_shape=[
            jax.ShapeDtypeStruct((_EP, _H), f32),
            jax.ShapeDtypeStruct((32, 1, _EP), f32),
        ],
    )(ea, Pr, Qc,
      eep['W1'], _rep(eep['b1']), _rep(eep['g1']), _rep(eep['be1']),
      eep['W2'], _rep(eep['b2']), _rep(eep['g2']), _rep(eep['be2']),
      lp['eW1'][2 * _H:], _rep(lp['eg1']), _rep(lp['ebe1']),
      lp['eW2'], _rep(lp['eb2']), _rep(lp['eg2']), _rep(lp['ebe2']))


def _edge_rest_call(es, Pr, Qc, lp):
    def body(es_ref, pr_ref, qc_ref, C_ref, lg1_ref, lbe1_ref,
             lW2_ref, lb2_ref, lg2_ref, lbe2_ref, o_ref, pk_ref):
        e1 = _edge_core(es_ref[...], pr_ref[...], qc_ref[...], C_ref[...],
                        lg1_ref[...], lbe1_ref[...], lW2_ref[...],
                        lb2_ref[...], lg2_ref[...], lbe2_ref[...])
        o_ref[...] = e1
        pk_ref[:, 0, :] = _pack_pairs(e1)

    c0 = lambda i: (0, 0)
    blk = lambda i: (i, 0)
    return pl.pallas_call(
        body,
        grid=(_EP // _TE,),
        in_specs=[
            pl.BlockSpec((_TE, _H), blk),
            pl.BlockSpec((_TE, 2 * _H), blk),
            pl.BlockSpec((_TE, 2 * _H), blk),
            pl.BlockSpec((_H, 2 * _H), c0),
            pl.BlockSpec((1, 2 * _H), c0),
            pl.BlockSpec((1, 2 * _H), c0),
            pl.BlockSpec((2 * _H, _H), c0),
            pl.BlockSpec((1, _H), c0),
            pl.BlockSpec((1, _H), c0),
            pl.BlockSpec((1, _H), c0),
        ],
        out_specs=[
            pl.BlockSpec((_TE, _H), blk),
            pl.BlockSpec((32, 1, _TE), lambda i: (0, 0, i)),
        ],
        out_shape=[
            jax.ShapeDtypeStruct((_EP, _H), f32),
            jax.ShapeDtypeStruct((32, 1, _EP), f32),
        ],
    )(es, Pr, Qc, lp['eW1'][2 * _H:], _rep(lp['eg1']), _rep(lp['ebe1']),
      lp['eW2'], _rep(lp['eb2']), _rep(lp['eg2']), _rep(lp['ebe2']))


def _node_core(x, agg_ref, cnt_ref, W1, b1, g1, be1, W2, b2, g2, be2):
    c = jnp.sum(cnt_ref[:, 0, :], axis=0, keepdims=True)
    inv = 1.0 / jnp.maximum(c, 1.0)
    agg = (agg_ref[:, 0, :] * inv).T
    H = (jnp.dot(x, W1[:_H], preferred_element_type=f32)
         + jnp.dot(agg, W1[_H:], preferred_element_type=f32) + b1)
    H = _geluk(_lnk(H, g1, be1))
    M = _lnk(jnp.dot(H, W2, preferred_element_type=f32) + b2, g2, be2)
    return x + M


_NODE_SPECS = [
    pl.BlockSpec((_TN, _H), lambda i: (i, 0)),
    pl.BlockSpec((_H, 1, _TN), lambda i: (0, 0, i)),
    pl.BlockSpec((32, 1, _TN), lambda i: (0, 0, i)),
    pl.BlockSpec((2 * _H, 2 * _H), lambda i: (0, 0)),
    pl.BlockSpec((1, 2 * _H), lambda i: (0, 0)),
    pl.BlockSpec((1, 2 * _H), lambda i: (0, 0)),
    pl.BlockSpec((1, 2 * _H), lambda i: (0, 0)),
    pl.BlockSpec((2 * _H, _H), lambda i: (0, 0)),
    pl.BlockSpec((1, _H), lambda i: (0, 0)),
    pl.BlockSpec((1, _H), lambda i: (0, 0)),
    pl.BlockSpec((1, _H), lambda i: (0, 0)),
]


def _node_mid_call(x, agg, cnt, lp, A, B, lb1):
    def body(x_ref, agg_ref, cnt_ref, W1_ref, b1_ref, g1_ref, be1_ref,
             W2_ref, b2_ref, g2_ref, be2_ref, A_ref, B_ref, lb1_ref,
             o_ref, p_ref, q_ref):
        xn = _node_core(x_ref[...], agg_ref, cnt_ref,
                        W1_ref[...], b1_ref[...], g1_ref[...],
                        be1_ref[...], W2_ref[...], b2_ref[...],
                        g2_ref[...], be2_ref[...])
        o_ref[...] = xn
        P, Q = _pq(xn, A_ref, B_ref, lb1_ref)
        p_ref[...] = P
        q_ref[...] = Q

    c0 = lambda i: (0, 0)
    blk = lambda i: (i, 0)
    return pl.pallas_call(
        body,
        grid=((_N + _TN - 1) // _TN,),
        in_specs=_NODE_SPECS + [
            pl.BlockSpec((_H, 2 * _H), c0),
            pl.BlockSpec((_H, 2 * _H), c0),
            pl.BlockSpec((1, 2 * _H), c0),
        ],
        out_specs=[
            pl.BlockSpec((_TN, _H), blk),
            pl.BlockSpec((_TN, 2 * _H), blk),
            pl.BlockSpec((_TN, 2 * _H), blk),
        ],
        out_shape=[
            jax.ShapeDtypeStruct((_N, _H), f32),
            jax.ShapeDtypeStruct((_N, 2 * _H), f32),
            jax.ShapeDtypeStruct((_N, 2 * _H), f32),
        ],
    )(x, agg, cnt, lp['nW1'], _rep(lp['nb1']), _rep(lp['ng1']),
      _rep(lp['nbe1']), lp['nW2'], _rep(lp['nb2']), _rep(lp['ng2']),
      _rep(lp['nbe2']), A, B, lb1)


def _node_last_call(x, agg, cnt, lp, dec):
    def body(x_ref, agg_ref, cnt_ref, W1_ref, b1_ref, g1_ref, be1_ref,
             W2_ref, b2_ref, g2_ref, be2_ref,
             dW1_ref, db1_ref, dW2_ref, db2_ref, o_ref):
        xn = _node_core(x_ref[...], agg_ref, cnt_ref,
                        W1_ref[...], b1_ref[...], g1_ref[...], be1_ref[...],
                        W2_ref[...], b2_ref[...], g2_ref[...], be2_ref[...])
        hd = _geluk(jnp.dot(xn, dW1_ref[...], preferred_element_type=f32)
                    + db1_ref[...])
        o_ref[...] = (jnp.dot(hd, dW2_ref[...], preferred_element_type=f32)
                      + db2_ref[...])

    c0 = lambda i: (0, 0)
    blk = lambda i: (i, 0)
    return pl.pallas_call(
        body,
        grid=((_N + _TN - 1) // _TN,),
        in_specs=_NODE_SPECS + [
            pl.BlockSpec((_H, _H), c0),
            pl.BlockSpec((1, _H), c0),
            pl.BlockSpec((_H, 4), c0),
            pl.BlockSpec((1, 4), c0),
        ],
        out_specs=pl.BlockSpec((_TN, 4), blk),
        out_shape=jax.ShapeDtypeStruct((_N, 4), f32),
    )(x, agg, cnt, lp['nW1'], _rep(lp['nb1']), _rep(lp['ng1']),
      _rep(lp['nbe1']), lp['nW2'], _rep(lp['nb2']), _rep(lp['ng2']),
      _rep(lp['nbe2']), dec['W1'], _rep(dec['b1']), dec['W2'], _rep(dec['b2']))


_mesh = plsc.VectorSubcoreMesh(core_axis_name="c", subcore_axis_name="s")


@functools.partial(
    pl.kernel,
    mesh=_mesh,
    out_type=(jax.ShapeDtypeStruct((_EP, 2 * _H), f32),
              jax.ShapeDtypeStruct((_EP, 2 * _H), f32)),
    scratch_types=[
        pltpu.VMEM((1, _GW), jnp.int32),
        pltpu.VMEM((2, _GW, 2 * _H), f32),
        pltpu.SemaphoreType.DMA,
        pltpu.SemaphoreType.DMA,
    ],
)
def _sc_gather(p_hbm, q_hbm, row_hbm, col_hbm, pr_out, qc_out,
               idx_v, rows_v, gsem, wsem):
    cid = lax.axis_index("c")
    sid = lax.axis_index("s")
    wid = sid * _NC + cid
    nit = _NST // _NW

    def do_table(idx3, table, out_hbm):
        def body(k, carry):
            t = k * _NW + wid
            par = lax.rem(k, 2)
            pltpu.sync_copy(idx3.at[t], idx_v)
            pltpu.async_copy(table.at[idx_v.at[0]], rows_v.at[par],
                             gsem).wait()

            @pl.when(k >= 1)
            def _():
                pltpu.make_async_copy(
                    rows_v.at[1 - par],
                    out_hbm.at[pl.ds(0, _GW)], wsem).wait()
            pltpu.async_copy(rows_v.at[par],
                             out_hbm.at[pl.ds(t * _GW, _GW)], wsem)
            return carry
        lax.fori_loop(0, nit, body, 0)
        pltpu.make_async_copy(rows_v.at[0], out_hbm.at[pl.ds(0, _GW)],
                              wsem).wait()

    do_table(row_hbm, p_hbm, pr_out)
    do_table(col_hbm, q_hbm, qc_out)


@functools.partial(
    pl.kernel,
    mesh=_mesh,
    compiler_params=pltpu.CompilerParams(needs_layout_passes=False),
    out_type=jax.ShapeDtypeStruct((_H, 1, _NPAD), f32),
    scratch_types=[
        pltpu.VMEM((_CH,), jnp.int32),
        pltpu.VMEM((_CH,), f32),
        pltpu.VMEM((2 * _NPAD,), f32),
    ],
)
def _sc_scatter(pk_hbm, col_hbm, agg_out, col_v, pk_v, acc):
    cid = lax.axis_index("c")
    sid = lax.axis_index("s")
    wid = sid * _NC + cid

    zv = jnp.zeros((16,), f32)

    def zb(i, carry):
        acc[pl.ds(pl.multiple_of(i * 16, 16), 16)] = zv
        return carry
    lax.fori_loop(0, (2 * _NPAD) // 16, zb, 0)

    def chunk(c, carry):
        pltpu.sync_copy(col_hbm.at[pl.ds(c * _CH, _CH)], col_v)
        pltpu.sync_copy(pk_hbm.at[wid, 0, pl.ds(c * _CH, _CH)], pk_v)

        def step(i, carry2):
            for u in range(8):
                base = pl.multiple_of((i * 8 + u) * 16, 16)
                cv = col_v[pl.ds(base, 16)]
                pb = plsc.bitcast(pk_v[pl.ds(base, 16)], jnp.int32)
                a = plsc.bitcast(pb & jnp.int32(-65536), f32)
                b = plsc.bitcast(pb << 16, f32)
                plsc.addupdate_scatter(acc, [cv], a)
                plsc.addupdate_scatter(acc, [cv + _NPAD], b)
            return carry2
        lax.fori_loop(0, _CH // 128, step, 0)
        return carry
    lax.fori_loop(0, _NCH, chunk, 0)

    pltpu.sync_copy(acc.at[pl.ds(0, _NPAD)],
                    agg_out.at[wid, 0, pl.ds(0, _NPAD)])
    pltpu.sync_copy(acc.at[pl.ds(_NPAD, _NPAD)],
                    agg_out.at[wid + 32, 0, pl.ds(0, _NPAD)])


@functools.partial(
    pl.kernel,
    mesh=_mesh,
    compiler_params=pltpu.CompilerParams(needs_layout_passes=False),
    out_type=jax.ShapeDtypeStruct((_NW, 1, _NPAD), f32),
    scratch_types=[
        pltpu.VMEM((_CH,), jnp.int32),
        pltpu.VMEM((_NPAD,), f32),
    ],
)
def _sc_count(col_hbm, cnt_out, col_v, acc):
    cid = lax.axis_index("c")
    sid = lax.axis_index("s")
    wid = sid * _NC + cid

    zv = jnp.zeros((16,), f32)
    ov = jnp.full((16,), 1.0, f32)

    def zb(i, carry):
        acc[pl.ds(pl.multiple_of(i * 16, 16), 16)] = zv
        return carry
    lax.fori_loop(0, _NPAD // 16, zb, 0)

    nch = _NCH // _NW

    def chunk(c, carry):
        pltpu.sync_copy(
            col_hbm.at[pl.ds((wid * nch + c) * _CH, _CH)], col_v)

        def step(i, carry2):
            for u in range(8):
                base = pl.multiple_of((i * 8 + u) * 16, 16)
                cv = col_v[pl.ds(base, 16)]
                plsc.addupdate_scatter(acc, [cv], ov)
            return carry2
        lax.fori_loop(0, _CH // 128, step, 0)
        return carry
    lax.fori_loop(0, nch, chunk, 0)

    pltpu.sync_copy(acc, cnt_out.at[wid, 0, pl.ds(0, _NPAD)])


def kernel(x, edge_attr, params, edge_index):
    npad = _EP - _E
    row3 = jnp.pad(edge_index[0], (0, npad)).reshape(_NST, 1, _GW)
    col3 = jnp.pad(edge_index[1], (0, npad)).reshape(_NST, 1, _GW)
    colp = jnp.pad(edge_index[1], (0, npad), constant_values=_N)
    xin = jnp.pad(x, ((0, 0), (0, 3)))
    ea = jnp.pad(edge_attr, ((0, npad), (0, 4)))

    lps = params['layers']
    enc = params['enc']
    x0, P, Q = _enc_call(xin, jnp.pad(enc['W'], ((0, 3), (0, 0))),
                         _rep(enc['b']), _rep(enc['g']), _rep(enc['be']),
                         lps[0]['eW1'][:_H], lps[0]['eW1'][_H:2 * _H],
                         _rep(lps[0]['eb1']))

    cnt = _sc_count(colp)

    ee = params['ee']
    eep = dict(ee)
    eep['W1'] = jnp.pad(ee['W1'], ((0, 4), (0, 0)))

    xcur = x0
    es = None
    out = None
    for li, lp in enumerate(lps):
        Pr, Qc = _sc_gather(P, Q, row3, col3)
        if li == 0:
            es, epk = _edge_first_call(ea, Pr, Qc, eep, lp)
        else:
            es, epk = _edge_rest_call(es, Pr, Qc, lp)
        agg = _sc_scatter(epk, colp)
        if li < 3:
            nxt = lps[li + 1]
            xcur, P, Q = _node_mid_call(xcur, agg, cnt, lp,
                                        nxt['eW1'][:_H],
                                        nxt['eW1'][_H:2 * _H],
                                        _rep(nxt['eb1']))
        else:
            out = _node_last_call(xcur, agg, cnt, lp, params['dec'])
    return out
